# Initial kernel scaffold; baseline (speedup 1.0000x reference)
import jax
import jax.numpy as jnp
from jax.experimental import pallas as pl


def kernel(features, edge_index, edge_weight, W1, b1, W2, b2):
    raise NotImplementedError("write your pallas kernel here")


# speedup vs baseline: 7.3172x; 7.3172x over previous
import functools

import jax
import jax.numpy as jnp
from jax import lax
from jax.experimental import pallas as pl
from jax.experimental.pallas import tpu as pltpu
from jax.experimental.pallas import tpu_sc as plsc

NC = 2
NS = 16
NW = NC * NS
L = 16
CHUNK = 128


def _bcast_lane(v, l):
    idx = jnp.full((L, 1), l, jnp.int32)
    dn = lax.GatherDimensionNumbers(
        offset_dims=(), collapsed_slice_dims=(0,), start_index_map=(0,))
    return lax.gather(v, idx, dn, slice_sizes=(1,),
                      mode=lax.GatherScatterMode.PROMISE_IN_BOUNDS)


def _sc_mesh():
    return plsc.VectorSubcoreMesh(
        core_axis_name="c", subcore_axis_name="s", num_cores=NC, num_subcores=NS)


def _sc_degree(col2d, ew2d, n_pad):
    ec = col2d.shape[0]
    rows_w = ec // NW
    npw = n_pad // NS

    def body(col_hbm, ew_hbm, out_hbm, colbuf, ewbuf, zbuf, acc):
        cid = lax.axis_index("c")
        sid = lax.axis_index("s")
        wid = cid * NS + sid

        def zero_body(i, _):
            zbuf[pl.ds(i * L, L)] = jnp.zeros((L,), jnp.float32)
            return 0
        lax.fori_loop(0, npw // L, zero_body, 0)
        pltpu.sync_copy(zbuf, acc.at[pl.ds(sid * npw, npw)])
        plsc.subcore_barrier()

        pltpu.sync_copy(col_hbm.at[pl.ds(wid * rows_w, rows_w)], colbuf)
        pltpu.sync_copy(ew_hbm.at[pl.ds(wid * rows_w, rows_w)], ewbuf)

        def chunk_body(j, _):
            pltpu.sync_copy(ewbuf.at[j], acc.at[colbuf.at[j]], add=True)
            return 0
        lax.fori_loop(0, rows_w, chunk_body, 0)
        plsc.subcore_barrier()
        pltpu.sync_copy(acc.at[pl.ds(sid * npw, npw)],
                        out_hbm.at[pl.ds(cid * n_pad + sid * npw, npw)])

    f = pl.kernel(
        body,
        out_type=jax.ShapeDtypeStruct((NC * n_pad,), jnp.float32),
        mesh=_sc_mesh(),
        scratch_types=[
            pltpu.VMEM((rows_w, CHUNK), jnp.int32),
            pltpu.VMEM((rows_w, CHUNK), jnp.float32),
            pltpu.VMEM((npw,), jnp.float32),
            pltpu.VMEM_SHARED((n_pad,), jnp.float32),
        ],
        name="sc_degree",
    )
    return f(col2d, ew2d)


def _sc_spmm(xs, row2d, col2d, ew2d, n_pad):
    d = xs.shape[1]
    g = d // L
    ec = row2d.shape[0]
    rows_w = ec // NW
    nrw = n_pad // NS

    def body(xs_hbm, row_hbm, col_hbm, ew_hbm, out_hbm,
             rowbuf, colbuf, ewbuf, gbuf, acc):
        cid = lax.axis_index("c")
        sid = lax.axis_index("s")
        wid = cid * NS + sid

        def zero_body(i, _):
            for t in range(g):
                gbuf[i, pl.ds(t * L, L)] = jnp.zeros((L,), jnp.float32)
            return 0
        lax.fori_loop(0, CHUNK, zero_body, 0)
        for k in range(nrw // CHUNK):
            pltpu.sync_copy(gbuf, acc.at[pl.ds(sid * nrw + k * CHUNK, CHUNK)])
        plsc.subcore_barrier()

        pltpu.sync_copy(row_hbm.at[pl.ds(wid * rows_w, rows_w)], rowbuf)
        pltpu.sync_copy(col_hbm.at[pl.ds(wid * rows_w, rows_w)], colbuf)
        pltpu.sync_copy(ew_hbm.at[pl.ds(wid * rows_w, rows_w)], ewbuf)

        def chunk_body(j, _):
            pltpu.sync_copy(xs_hbm.at[rowbuf.at[j]], gbuf)

            def group_body(q, _):
                ewv = ewbuf[j, pl.ds(q * L, L)]
                for l in range(L):
                    e = q * L + l
                    s = _bcast_lane(ewv, l)
                    for t in range(g):
                        gbuf[e, pl.ds(t * L, L)] = gbuf[e, pl.ds(t * L, L)] * s
                return 0
            lax.fori_loop(0, CHUNK // L, group_body, 0)

            pltpu.sync_copy(gbuf, acc.at[colbuf.at[j]], add=True)
            return 0
        lax.fori_loop(0, rows_w, chunk_body, 0)
        plsc.subcore_barrier()
        pltpu.sync_copy(acc.at[pl.ds(sid * nrw, nrw)],
                        out_hbm.at[cid, pl.ds(sid * nrw, nrw)])

    f = pl.kernel(
        body,
        out_type=jax.ShapeDtypeStruct((NC, n_pad, d), jnp.float32),
        mesh=_sc_mesh(),
        scratch_types=[
            pltpu.VMEM((rows_w, CHUNK), jnp.int32),
            pltpu.VMEM((rows_w, CHUNK), jnp.int32),
            pltpu.VMEM((rows_w, CHUNK), jnp.float32),
            pltpu.VMEM((CHUNK, d), jnp.float32),
            pltpu.VMEM_SHARED((n_pad, d), jnp.float32),
        ],
        name=f"sc_spmm_d{d}",
    )
    return f(xs, row2d, col2d, ew2d)


def _tc_prep(features, w1, dega, degb, br=400):
    n, nf = features.shape
    nh = w1.shape[1]

    def body(f_ref, w_ref, d0_ref, d1_ref, xs_ref, dinv_ref):
        dinv = lax.rsqrt(1.0 + d0_ref[...] + d1_ref[...])
        x1 = jnp.dot(f_ref[...], w_ref[...], preferred_element_type=jnp.float32)
        xs_ref[...] = x1 * dinv
        dinv_ref[...] = dinv

    return pl.pallas_call(
        body,
        grid=(n // br,),
        in_specs=[
            pl.BlockSpec((br, nf), lambda i: (i, 0)),
            pl.BlockSpec((nf, nh), lambda i: (0, 0)),
            pl.BlockSpec((br, 1), lambda i: (i, 0)),
            pl.BlockSpec((br, 1), lambda i: (i, 0)),
        ],
        out_specs=[
            pl.BlockSpec((br, nh), lambda i: (i, 0)),
            pl.BlockSpec((br, 1), lambda i: (i, 0)),
        ],
        out_shape=[
            jax.ShapeDtypeStruct((n, nh), jnp.float32),
            jax.ShapeDtypeStruct((n, 1), jnp.float32),
        ],
        name="tc_prep",
    )(features, w1, dega, degb)


def _tc_mid(s1, xs1, dinv, b1, br=400):
    n, nh = xs1.shape

    def body(s_ref, xs_ref, dv_ref, b_ref, o_ref):
        s = s_ref[0] + s_ref[1] + xs_ref[...]
        h = jnp.maximum(dv_ref[...] * s + b_ref[...], 0.0)
        o_ref[...] = h * dv_ref[...]

    return pl.pallas_call(
        body,
        grid=(n // br,),
        in_specs=[
            pl.BlockSpec((2, br, nh), lambda i: (0, i, 0)),
            pl.BlockSpec((br, nh), lambda i: (i, 0)),
            pl.BlockSpec((br, 1), lambda i: (i, 0)),
            pl.BlockSpec((1, nh), lambda i: (0, 0)),
        ],
        out_specs=pl.BlockSpec((br, nh), lambda i: (i, 0)),
        out_shape=jax.ShapeDtypeStruct((n, nh), jnp.float32),
        name="tc_mid",
    )(s1, xs1, dinv, b1)


def _tc_final(s2, hs, dinv, w2, b2, br=400):
    n, nh = hs.shape
    ncls = w2.shape[1]

    def body(s_ref, hs_ref, dv_ref, w_ref, b_ref, o_ref):
        z = dv_ref[...] * (s_ref[0] + s_ref[1] + hs_ref[...])
        o = jnp.dot(z, w_ref[...], preferred_element_type=jnp.float32) + b_ref[...]
        m = jnp.max(o, axis=1, keepdims=True)
        lse = jnp.log(jnp.sum(jnp.exp(o - m), axis=1, keepdims=True))
        o_ref[...] = o - m - lse

    return pl.pallas_call(
        body,
        grid=(n // br,),
        in_specs=[
            pl.BlockSpec((2, br, nh), lambda i: (0, i, 0)),
            pl.BlockSpec((br, nh), lambda i: (i, 0)),
            pl.BlockSpec((br, 1), lambda i: (i, 0)),
            pl.BlockSpec((nh, ncls), lambda i: (0, 0)),
            pl.BlockSpec((1, ncls), lambda i: (0, 0)),
        ],
        out_specs=pl.BlockSpec((br, ncls), lambda i: (i, 0)),
        out_shape=jax.ShapeDtypeStruct((n, ncls), jnp.float32),
        name="tc_final",
    )(s2, hs, dinv, w2, b2)


@jax.jit
def kernel(features, edge_index, edge_weight, W1, b1, W2, b2):
    n, nf = features.shape
    nh = W1.shape[1]
    ncls = W2.shape[1]
    e = edge_index.shape[1]

    rows_w = -(-(-(-e // (NW * CHUNK))) // 8) * 8
    e_pad = rows_w * NW * CHUNK
    pad = e_pad - e
    row = jnp.concatenate([edge_index[0], jnp.zeros((pad,), edge_index.dtype)])
    col = jnp.concatenate([edge_index[1], jnp.zeros((pad,), edge_index.dtype)])
    ew = jnp.concatenate([edge_weight, jnp.zeros((pad,), edge_weight.dtype)])
    row2d = row.reshape(-1, CHUNK)
    col2d = col.reshape(-1, CHUNK)
    ew2d = ew.reshape(-1, CHUNK)

    n_pad = -(-n // (NS * CHUNK)) * (NS * CHUNK)

    degp = _sc_degree(col2d, ew2d, n_pad).reshape(NC, n_pad)
    dega = degp[0, :n].reshape(n, 1)
    degb = degp[1, :n].reshape(n, 1)

    xs1, dinv = _tc_prep(features, W1, dega, degb)
    s1 = _sc_spmm(xs1, row2d, col2d, ew2d, n_pad)

    hs = _tc_mid(s1, xs1, dinv, b1.reshape(1, nh))
    s2 = _sc_spmm(hs, row2d, col2d, ew2d, n_pad)
    return _tc_final(s2, hs, dinv, W2, b2.reshape(1, ncls))


# speedup vs baseline: 7.5194x; 1.0276x over previous
import functools

import jax
import jax.numpy as jnp
from jax import lax
from jax.experimental import pallas as pl
from jax.experimental.pallas import tpu as pltpu
from jax.experimental.pallas import tpu_sc as plsc

NC = 2
NS = 16
NW = NC * NS
L = 16
CHUNK = 128


def _bcast_lane(v, l):
    idx = jnp.full((L, 1), l, jnp.int32)
    dn = lax.GatherDimensionNumbers(
        offset_dims=(), collapsed_slice_dims=(0,), start_index_map=(0,))
    return lax.gather(v, idx, dn, slice_sizes=(1,),
                      mode=lax.GatherScatterMode.PROMISE_IN_BOUNDS)


def _sc_mesh():
    return plsc.VectorSubcoreMesh(
        core_axis_name="c", subcore_axis_name="s", num_cores=NC, num_subcores=NS)


def _sc_degree(col2d, ew2d, n_pad):
    ec = col2d.shape[0]
    rows_w = ec // NW
    npw = n_pad // NS

    def body(col_hbm, ew_hbm, out_hbm, colbuf, ewbuf, zbuf, acc):
        cid = lax.axis_index("c")
        sid = lax.axis_index("s")
        wid = cid * NS + sid

        def zero_body(i, _):
            zbuf[pl.ds(i * L, L)] = jnp.zeros((L,), jnp.float32)
            return 0
        lax.fori_loop(0, npw // L, zero_body, 0)
        pltpu.sync_copy(zbuf, acc.at[pl.ds(sid * npw, npw)])
        plsc.subcore_barrier()

        pltpu.sync_copy(col_hbm.at[pl.ds(wid * rows_w, rows_w)], colbuf)
        pltpu.sync_copy(ew_hbm.at[pl.ds(wid * rows_w, rows_w)], ewbuf)

        def chunk_body(j, _):
            pltpu.sync_copy(ewbuf.at[j], acc.at[colbuf.at[j]], add=True)
            return 0
        lax.fori_loop(0, rows_w, chunk_body, 0)
        plsc.subcore_barrier()
        pltpu.sync_copy(acc.at[pl.ds(sid * npw, npw)],
                        out_hbm.at[pl.ds(cid * n_pad + sid * npw, npw)])

    f = pl.kernel(
        body,
        out_type=jax.ShapeDtypeStruct((NC * n_pad,), jnp.float32),
        mesh=_sc_mesh(),
        scratch_types=[
            pltpu.VMEM((rows_w, CHUNK), jnp.int32),
            pltpu.VMEM((rows_w, CHUNK), jnp.float32),
            pltpu.VMEM((npw,), jnp.float32),
            pltpu.VMEM_SHARED((n_pad,), jnp.float32),
        ],
        name="sc_degree",
    )
    return f(col2d, ew2d)


def _sc_spmm(xs, row2d, col2d, ew2d, n_pad):
    d = xs.shape[1]
    g = d // L
    ec = row2d.shape[0]
    rows_w = ec // NW
    nrw = n_pad // NS

    sb = 8
    nsb = rows_w // sb

    def body(xs_hbm, row_hbm, col_hbm, ew_hbm, out_hbm,
             erow, ecol, eew, gbuf, gsem, ssem, esem, acc):
        cid = lax.axis_index("c")
        sid = lax.axis_index("s")
        wid = cid * NS + sid

        def zero_body(i, _):
            for t in range(g):
                gbuf[0, i, pl.ds(t * L, L)] = jnp.zeros((L,), jnp.float32)
            return 0
        lax.fori_loop(0, CHUNK, zero_body, 0)
        for k in range(nrw // CHUNK):
            pltpu.sync_copy(gbuf.at[0],
                            acc.at[pl.ds(sid * nrw + k * CHUNK, CHUNK)])
        plsc.subcore_barrier()

        def eload(si, slot, sem):
            base = wid * rows_w + si * sb
            pltpu.async_copy(row_hbm.at[pl.ds(base, sb)], erow.at[slot], sem)
            pltpu.async_copy(col_hbm.at[pl.ds(base, sb)], ecol.at[slot], sem)
            pltpu.async_copy(ew_hbm.at[pl.ds(base, sb)], eew.at[slot], sem)

        def ewait(si, slot, sem):
            base = wid * rows_w + si * sb
            pltpu.make_async_copy(
                row_hbm.at[pl.ds(base, sb)], erow.at[slot], sem).wait()
            pltpu.make_async_copy(
                col_hbm.at[pl.ds(base, sb)], ecol.at[slot], sem).wait()
            pltpu.make_async_copy(
                ew_hbm.at[pl.ds(base, sb)], eew.at[slot], sem).wait()

        eload(0, 0, esem)
        ewait(0, 0, esem)
        pltpu.async_copy(xs_hbm.at[erow.at[0, 0]], gbuf.at[0], gsem)

        def chunk_body(j, _):
            b = lax.rem(j, 2)
            nb = 1 - b
            si = lax.div(j, sb)
            k = lax.rem(j, sb)
            slot = lax.rem(si, 2)

            pltpu.make_async_copy(
                xs_hbm.at[erow.at[slot, k]], gbuf.at[b], gsem).wait()

            @pl.when(j >= 1)
            def _():
                jp = j - 1
                pltpu.make_async_copy(
                    gbuf.at[nb],
                    acc.at[ecol.at[lax.rem(lax.div(jp, sb), 2),
                                   lax.rem(jp, sb)]],
                    ssem).wait()

            @pl.when((k == 0) & (si + 1 < nsb))
            def _():
                eload(si + 1, 1 - slot, esem)

            @pl.when(j + 1 < rows_w)
            def _():
                jn = j + 1
                sin = lax.div(jn, sb)
                slotn = lax.rem(sin, 2)

                @pl.when(k == sb - 1)
                def _():
                    ewait(sin, slotn, esem)
                pltpu.async_copy(
                    xs_hbm.at[erow.at[slotn, lax.rem(jn, sb)]],
                    gbuf.at[nb], gsem)

            def group_body(q, _):
                ewv = eew[slot, k, pl.ds(q * L, L)]
                for l in range(L):
                    e = q * L + l
                    s = _bcast_lane(ewv, l)
                    for t in range(g):
                        gbuf[b, e, pl.ds(t * L, L)] = (
                            gbuf[b, e, pl.ds(t * L, L)] * s)
                return 0
            lax.fori_loop(0, CHUNK // L, group_body, 0)

            pltpu.async_copy(gbuf.at[b], acc.at[ecol.at[slot, k]],
                             ssem, add=True)
            return 0
        lax.fori_loop(0, rows_w, chunk_body, 0)

        jl = rows_w - 1
        pltpu.make_async_copy(
            gbuf.at[lax.rem(jl, 2)],
            acc.at[ecol.at[lax.rem(lax.div(jl, sb), 2), lax.rem(jl, sb)]],
            ssem).wait()
        plsc.subcore_barrier()
        pltpu.sync_copy(acc.at[pl.ds(sid * nrw, nrw)],
                        out_hbm.at[cid, pl.ds(sid * nrw, nrw)])

    f = pl.kernel(
        body,
        out_type=jax.ShapeDtypeStruct((NC, n_pad, d), jnp.float32),
        mesh=_sc_mesh(),
        scratch_types=[
            pltpu.VMEM((2, sb, CHUNK), jnp.int32),
            pltpu.VMEM((2, sb, CHUNK), jnp.int32),
            pltpu.VMEM((2, sb, CHUNK), jnp.float32),
            pltpu.VMEM((2, CHUNK, d), jnp.float32),
            pltpu.SemaphoreType.DMA,
            pltpu.SemaphoreType.DMA,
            pltpu.SemaphoreType.DMA,
            pltpu.VMEM_SHARED((n_pad, d), jnp.float32),
        ],
        name=f"sc_spmm_d{d}",
    )
    return f(xs, row2d, col2d, ew2d)


def _tc_prep(features, w1, dega, degb, br=400):
    n, nf = features.shape
    nh = w1.shape[1]

    def body(f_ref, w_ref, d0_ref, d1_ref, xs_ref, dinv_ref):
        dinv = lax.rsqrt(1.0 + d0_ref[...] + d1_ref[...])
        x1 = jnp.dot(f_ref[...], w_ref[...], preferred_element_type=jnp.float32)
        xs_ref[...] = x1 * dinv
        dinv_ref[...] = dinv

    return pl.pallas_call(
        body,
        grid=(n // br,),
        in_specs=[
            pl.BlockSpec((br, nf), lambda i: (i, 0)),
            pl.BlockSpec((nf, nh), lambda i: (0, 0)),
            pl.BlockSpec((br, 1), lambda i: (i, 0)),
            pl.BlockSpec((br, 1), lambda i: (i, 0)),
        ],
        out_specs=[
            pl.BlockSpec((br, nh), lambda i: (i, 0)),
            pl.BlockSpec((br, 1), lambda i: (i, 0)),
        ],
        out_shape=[
            jax.ShapeDtypeStruct((n, nh), jnp.float32),
            jax.ShapeDtypeStruct((n, 1), jnp.float32),
        ],
        name="tc_prep",
    )(features, w1, dega, degb)


def _tc_mid(s1, xs1, dinv, b1, br=400):
    n, nh = xs1.shape

    def body(s_ref, xs_ref, dv_ref, b_ref, o_ref):
        s = s_ref[0] + s_ref[1] + xs_ref[...]
        h = jnp.maximum(dv_ref[...] * s + b_ref[...], 0.0)
        o_ref[...] = h * dv_ref[...]

    return pl.pallas_call(
        body,
        grid=(n // br,),
        in_specs=[
            pl.BlockSpec((2, br, nh), lambda i: (0, i, 0)),
            pl.BlockSpec((br, nh), lambda i: (i, 0)),
            pl.BlockSpec((br, 1), lambda i: (i, 0)),
            pl.BlockSpec((1, nh), lambda i: (0, 0)),
        ],
        out_specs=pl.BlockSpec((br, nh), lambda i: (i, 0)),
        out_shape=jax.ShapeDtypeStruct((n, nh), jnp.float32),
        name="tc_mid",
    )(s1, xs1, dinv, b1)


def _tc_final(s2, hs, dinv, w2, b2, br=400):
    n, nh = hs.shape
    ncls = w2.shape[1]

    def body(s_ref, hs_ref, dv_ref, w_ref, b_ref, o_ref):
        z = dv_ref[...] * (s_ref[0] + s_ref[1] + hs_ref[...])
        o = jnp.dot(z, w_ref[...], preferred_element_type=jnp.float32) + b_ref[...]
        m = jnp.max(o, axis=1, keepdims=True)
        lse = jnp.log(jnp.sum(jnp.exp(o - m), axis=1, keepdims=True))
        o_ref[...] = o - m - lse

    return pl.pallas_call(
        body,
        grid=(n // br,),
        in_specs=[
            pl.BlockSpec((2, br, nh), lambda i: (0, i, 0)),
            pl.BlockSpec((br, nh), lambda i: (i, 0)),
            pl.BlockSpec((br, 1), lambda i: (i, 0)),
            pl.BlockSpec((nh, ncls), lambda i: (0, 0)),
            pl.BlockSpec((1, ncls), lambda i: (0, 0)),
        ],
        out_specs=pl.BlockSpec((br, ncls), lambda i: (i, 0)),
        out_shape=jax.ShapeDtypeStruct((n, ncls), jnp.float32),
        name="tc_final",
    )(s2, hs, dinv, w2, b2)


@jax.jit
def kernel(features, edge_index, edge_weight, W1, b1, W2, b2):
    n, nf = features.shape
    nh = W1.shape[1]
    ncls = W2.shape[1]
    e = edge_index.shape[1]

    rows_w = -(-(-(-e // (NW * CHUNK))) // 8) * 8
    e_pad = rows_w * NW * CHUNK
    pad = e_pad - e
    row = jnp.concatenate([edge_index[0], jnp.zeros((pad,), edge_index.dtype)])
    col = jnp.concatenate([edge_index[1], jnp.zeros((pad,), edge_index.dtype)])
    ew = jnp.concatenate([edge_weight, jnp.zeros((pad,), edge_weight.dtype)])
    row2d = row.reshape(-1, CHUNK)
    col2d = col.reshape(-1, CHUNK)
    ew2d = ew.reshape(-1, CHUNK)

    n_pad = -(-n // (NS * CHUNK)) * (NS * CHUNK)

    degp = _sc_degree(col2d, ew2d, n_pad).reshape(NC, n_pad)
    dega = degp[0, :n].reshape(n, 1)
    degb = degp[1, :n].reshape(n, 1)

    xs1, dinv = _tc_prep(features, W1, dega, degb)
    s1 = _sc_spmm(xs1, row2d, col2d, ew2d, n_pad)

    hs = _tc_mid(s1, xs1, dinv, b1.reshape(1, nh))
    s2 = _sc_spmm(hs, row2d, col2d, ew2d, n_pad)
    return _tc_final(s2, hs, dinv, W2, b2.reshape(1, ncls))


# speedup vs baseline: 7.5212x; 1.0002x over previous
import functools

import jax
import jax.numpy as jnp
import numpy as np
from jax import lax
from jax.experimental import pallas as pl
from jax.experimental.pallas import tpu as pltpu
from jax.experimental.pallas import tpu_sc as plsc

NC = 2
NS = 16
NW = NC * NS
L = 16
CHUNK = 128


def _unpack_perm(d):
    p = np.empty((d,), np.int32)
    for t in range(d // 32):
        for i in range(16):
            p[t * 32 + 2 * i] = t * 32 + i
            p[t * 32 + 2 * i + 1] = t * 32 + 16 + i
    return p


def _bcast_lane(v, l):
    idx = jnp.full((L, 1), l, jnp.int32)
    dn = lax.GatherDimensionNumbers(
        offset_dims=(), collapsed_slice_dims=(0,), start_index_map=(0,))
    return lax.gather(v, idx, dn, slice_sizes=(1,),
                      mode=lax.GatherScatterMode.PROMISE_IN_BOUNDS)


def _sc_mesh():
    return plsc.VectorSubcoreMesh(
        core_axis_name="c", subcore_axis_name="s", num_cores=NC, num_subcores=NS)


def _sc_degree(col2d, ew2d, n_pad):
    ec = col2d.shape[0]
    rows_w = ec // NW
    npw = n_pad // NS

    def body(col_hbm, ew_hbm, out_hbm, colbuf, ewbuf, zbuf, acc):
        cid = lax.axis_index("c")
        sid = lax.axis_index("s")
        wid = cid * NS + sid

        def zero_body(i, _):
            zbuf[pl.ds(i * L, L)] = jnp.zeros((L,), jnp.float32)
            return 0
        lax.fori_loop(0, npw // L, zero_body, 0)
        pltpu.sync_copy(zbuf, acc.at[pl.ds(sid * npw, npw)])
        plsc.subcore_barrier()

        pltpu.sync_copy(col_hbm.at[pl.ds(wid * rows_w, rows_w)], colbuf)
        pltpu.sync_copy(ew_hbm.at[pl.ds(wid * rows_w, rows_w)], ewbuf)

        def chunk_body(j, _):
            pltpu.sync_copy(ewbuf.at[j], acc.at[colbuf.at[j]], add=True)
            return 0
        lax.fori_loop(0, rows_w, chunk_body, 0)
        plsc.subcore_barrier()
        pltpu.sync_copy(acc.at[pl.ds(sid * npw, npw)],
                        out_hbm.at[pl.ds(cid * n_pad + sid * npw, npw)])

    f = pl.kernel(
        body,
        out_type=jax.ShapeDtypeStruct((NC * n_pad,), jnp.float32),
        mesh=_sc_mesh(),
        scratch_types=[
            pltpu.VMEM((rows_w, CHUNK), jnp.int32),
            pltpu.VMEM((rows_w, CHUNK), jnp.float32),
            pltpu.VMEM((npw,), jnp.float32),
            pltpu.VMEM_SHARED((n_pad,), jnp.float32),
        ],
        name="sc_degree",
    )
    return f(col2d, ew2d)


def _sc_spmm(xs, row2d, col2d, ew2d, n_pad):
    d = xs.shape[1]
    ec = row2d.shape[0]
    rows_w = ec // NW
    nrw = n_pad // NS

    sb = 8
    nsb = rows_w // sb

    def body(xs_hbm, row_hbm, col_hbm, ew_hbm, out_hbm,
             erow, ecol, eew, gbuf, gsem, ssem, esem, acc):
        cid = lax.axis_index("c")
        sid = lax.axis_index("s")
        wid = cid * NS + sid

        def zero_body(i, _):
            for t in range(d // L):
                gbuf[0, i, pl.ds(t * L, L)] = jnp.zeros((L,), jnp.float32)
            return 0
        lax.fori_loop(0, CHUNK, zero_body, 0)
        for k in range(nrw // CHUNK):
            pltpu.sync_copy(gbuf.at[0],
                            acc.at[pl.ds(sid * nrw + k * CHUNK, CHUNK)])
        plsc.subcore_barrier()

        def eload(si, slot, sem):
            base = wid * rows_w + si * sb
            pltpu.async_copy(row_hbm.at[pl.ds(base, sb)], erow.at[slot], sem)
            pltpu.async_copy(col_hbm.at[pl.ds(base, sb)], ecol.at[slot], sem)
            pltpu.async_copy(ew_hbm.at[pl.ds(base, sb)], eew.at[slot], sem)

        def ewait(si, slot, sem):
            base = wid * rows_w + si * sb
            pltpu.make_async_copy(
                row_hbm.at[pl.ds(base, sb)], erow.at[slot], sem).wait()
            pltpu.make_async_copy(
                col_hbm.at[pl.ds(base, sb)], ecol.at[slot], sem).wait()
            pltpu.make_async_copy(
                ew_hbm.at[pl.ds(base, sb)], eew.at[slot], sem).wait()

        eload(0, 0, esem)
        ewait(0, 0, esem)
        pltpu.async_copy(xs_hbm.at[erow.at[0, 0]], gbuf.at[0], gsem)

        def chunk_body(j, _):
            b = lax.rem(j, 2)
            nb = 1 - b
            si = lax.div(j, sb)
            k = lax.rem(j, sb)
            slot = lax.rem(si, 2)

            pltpu.make_async_copy(
                xs_hbm.at[erow.at[slot, k]], gbuf.at[b], gsem).wait()

            @pl.when(j >= 1)
            def _():
                jp = j - 1
                pltpu.make_async_copy(
                    gbuf.at[nb],
                    acc.at[ecol.at[lax.rem(lax.div(jp, sb), 2),
                                   lax.rem(jp, sb)]],
                    ssem).wait()

            @pl.when((k == 0) & (si + 1 < nsb))
            def _():
                eload(si + 1, 1 - slot, esem)

            @pl.when(j + 1 < rows_w)
            def _():
                jn = j + 1
                sin = lax.div(jn, sb)
                slotn = lax.rem(sin, 2)

                @pl.when(k == sb - 1)
                def _():
                    ewait(sin, slotn, esem)

                pltpu.async_copy(
                    xs_hbm.at[erow.at[slotn, lax.rem(jn, sb)]],
                    gbuf.at[nb], gsem)

            def group_body(q, _):
                ewv = eew[slot, k, pl.ds(q * L, L)]
                for l in range(L):
                    e = q * L + l
                    s = _bcast_lane(ewv, l)
                    for t in range(d // L):
                        gbuf[b, e, pl.ds(t * L, L)] = (
                            gbuf[b, e, pl.ds(t * L, L)] * s)
                return 0
            lax.fori_loop(0, CHUNK // L, group_body, 0)

            pltpu.async_copy(gbuf.at[b], acc.at[ecol.at[slot, k]],
                             ssem, add=True)
            return 0
        lax.fori_loop(0, rows_w, chunk_body, 0)

        jl = rows_w - 1
        pltpu.make_async_copy(
            gbuf.at[lax.rem(jl, 2)],
            acc.at[ecol.at[lax.rem(lax.div(jl, sb), 2), lax.rem(jl, sb)]],
            ssem).wait()
        plsc.subcore_barrier()
        pltpu.sync_copy(acc.at[pl.ds(sid * nrw, nrw)],
                        out_hbm.at[cid, pl.ds(sid * nrw, nrw)])

    f = pl.kernel(
        body,
        out_type=jax.ShapeDtypeStruct((NC, n_pad, d), jnp.float32),
        mesh=_sc_mesh(),
        scratch_types=[
            pltpu.VMEM((2, sb, CHUNK), jnp.int32),
            pltpu.VMEM((2, sb, CHUNK), jnp.int32),
            pltpu.VMEM((2, sb, CHUNK), jnp.float32),
            pltpu.VMEM((2, CHUNK, d), jnp.float32),
            pltpu.SemaphoreType.DMA,
            pltpu.SemaphoreType.DMA,
            pltpu.SemaphoreType.DMA,
            pltpu.VMEM_SHARED((n_pad, d), jnp.float32),
        ],
        name=f"sc_spmm_d{d}",
    )
    return f(xs, row2d, col2d, ew2d)


def _tc_prep(features, w1, dega, degb, br=400):
    n, nf = features.shape
    nh = w1.shape[1]

    def body(f_ref, w_ref, d0_ref, d1_ref, xs_ref, dinv_ref):
        dinv = lax.rsqrt(1.0 + d0_ref[...] + d1_ref[...])
        x1 = jnp.dot(f_ref[...], w_ref[...], preferred_element_type=jnp.float32)
        xs_ref[...] = x1 * dinv
        dinv_ref[...] = dinv

    return pl.pallas_call(
        body,
        grid=(n // br,),
        in_specs=[
            pl.BlockSpec((br, nf), lambda i: (i, 0)),
            pl.BlockSpec((nf, nh), lambda i: (0, 0)),
            pl.BlockSpec((br, 1), lambda i: (i, 0)),
            pl.BlockSpec((br, 1), lambda i: (i, 0)),
        ],
        out_specs=[
            pl.BlockSpec((br, nh), lambda i: (i, 0)),
            pl.BlockSpec((br, 1), lambda i: (i, 0)),
        ],
        out_shape=[
            jax.ShapeDtypeStruct((n, nh), jnp.float32),
            jax.ShapeDtypeStruct((n, 1), jnp.float32),
        ],
        name="tc_prep",
    )(features, w1, dega, degb)


def _tc_mid(s1, xs1, dinv, b1, br=400):
    n, nh = xs1.shape

    def body(s_ref, xs_ref, dv_ref, b_ref, o_ref):
        s = s_ref[0] + s_ref[1] + xs_ref[...]
        h = jnp.maximum(dv_ref[...] * s + b_ref[...], 0.0)
        o_ref[...] = h * dv_ref[...]

    return pl.pallas_call(
        body,
        grid=(n // br,),
        in_specs=[
            pl.BlockSpec((2, br, nh), lambda i: (0, i, 0)),
            pl.BlockSpec((br, nh), lambda i: (i, 0)),
            pl.BlockSpec((br, 1), lambda i: (i, 0)),
            pl.BlockSpec((1, nh), lambda i: (0, 0)),
        ],
        out_specs=pl.BlockSpec((br, nh), lambda i: (i, 0)),
        out_shape=jax.ShapeDtypeStruct((n, nh), jnp.float32),
        name="tc_mid",
    )(s1, xs1, dinv, b1)


def _tc_final(s2, hs, dinv, w2, b2, br=400):
    n, nh = hs.shape
    ncls = w2.shape[1]

    def body(s_ref, hs_ref, dv_ref, w_ref, b_ref, o_ref):
        z = dv_ref[...] * (s_ref[0] + s_ref[1] + hs_ref[...])
        o = jnp.dot(z, w_ref[...], preferred_element_type=jnp.float32) + b_ref[...]
        m = jnp.max(o, axis=1, keepdims=True)
        lse = jnp.log(jnp.sum(jnp.exp(o - m), axis=1, keepdims=True))
        o_ref[...] = o - m - lse

    return pl.pallas_call(
        body,
        grid=(n // br,),
        in_specs=[
            pl.BlockSpec((2, br, nh), lambda i: (0, i, 0)),
            pl.BlockSpec((br, nh), lambda i: (i, 0)),
            pl.BlockSpec((br, 1), lambda i: (i, 0)),
            pl.BlockSpec((nh, ncls), lambda i: (0, 0)),
            pl.BlockSpec((1, ncls), lambda i: (0, 0)),
        ],
        out_specs=pl.BlockSpec((br, ncls), lambda i: (i, 0)),
        out_shape=jax.ShapeDtypeStruct((n, ncls), jnp.float32),
        name="tc_final",
    )(s2, hs, dinv, w2, b2)


@jax.jit
def kernel(features, edge_index, edge_weight, W1, b1, W2, b2):
    n, nf = features.shape
    nh = W1.shape[1]
    ncls = W2.shape[1]
    e = edge_index.shape[1]

    rows_w = -(-(-(-e // (NW * CHUNK))) // 8) * 8
    e_pad = rows_w * NW * CHUNK
    pad = e_pad - e
    row = jnp.concatenate([edge_index[0], jnp.zeros((pad,), edge_index.dtype)])
    col = jnp.concatenate([edge_index[1], jnp.zeros((pad,), edge_index.dtype)])
    ew = jnp.concatenate([edge_weight, jnp.zeros((pad,), edge_weight.dtype)])
    row2d = row.reshape(-1, CHUNK)
    col2d = col.reshape(-1, CHUNK)
    ew2d = ew.reshape(-1, CHUNK)

    n_pad = -(-n // (NS * CHUNK)) * (NS * CHUNK)

    degp = _sc_degree(col2d, ew2d, n_pad).reshape(NC, n_pad)
    dega = degp[0, :n].reshape(n, 1)
    degb = degp[1, :n].reshape(n, 1)

    xs1, dinv = _tc_prep(features, W1, dega, degb)
    s1 = _sc_spmm(xs1, row2d, col2d, ew2d, n_pad)

    hs = _tc_mid(s1, xs1, dinv, b1.reshape(1, nh))
    s2 = _sc_spmm(hs, row2d, col2d, ew2d, n_pad)
    return _tc_final(s2, hs, dinv, W2, b2.reshape(1, ncls))


# speedup vs baseline: 7.5233x; 1.0003x over previous
import functools

import jax
import jax.numpy as jnp
import numpy as np
from jax import lax
from jax.experimental import pallas as pl
from jax.experimental.pallas import tpu as pltpu
from jax.experimental.pallas import tpu_sc as plsc

NC = 2
NS = 16
NW = NC * NS
L = 16
CHUNK = 128


def _unpack_perm(d):
    p = np.empty((d,), np.int32)
    for t in range(d // 32):
        for i in range(16):
            p[t * 32 + 2 * i] = t * 32 + i
            p[t * 32 + 2 * i + 1] = t * 32 + 16 + i
    return p


def _bcast_lane(v, l):
    idx = jnp.full((L, 1), l, jnp.int32)
    dn = lax.GatherDimensionNumbers(
        offset_dims=(), collapsed_slice_dims=(0,), start_index_map=(0,))
    return lax.gather(v, idx, dn, slice_sizes=(1,),
                      mode=lax.GatherScatterMode.PROMISE_IN_BOUNDS)


def _sc_mesh():
    return plsc.VectorSubcoreMesh(
        core_axis_name="c", subcore_axis_name="s", num_cores=NC, num_subcores=NS)


def _sc_degree(col2d, ew2d, n_pad):
    ec = col2d.shape[0]
    rows_w = ec // NW
    npw = n_pad // NS

    def body(col_hbm, ew_hbm, out_hbm, colbuf, ewbuf, zbuf, acc):
        cid = lax.axis_index("c")
        sid = lax.axis_index("s")
        wid = cid * NS + sid

        def zero_body(i, _):
            zbuf[pl.ds(i * L, L)] = jnp.zeros((L,), jnp.float32)
            return 0
        lax.fori_loop(0, npw // L, zero_body, 0)
        pltpu.sync_copy(zbuf, acc.at[pl.ds(sid * npw, npw)])
        plsc.subcore_barrier()

        pltpu.sync_copy(col_hbm.at[pl.ds(wid * rows_w, rows_w)], colbuf)
        pltpu.sync_copy(ew_hbm.at[pl.ds(wid * rows_w, rows_w)], ewbuf)

        def chunk_body(j, _):
            pltpu.sync_copy(ewbuf.at[j], acc.at[colbuf.at[j]], add=True)
            return 0
        lax.fori_loop(0, rows_w, chunk_body, 0)
        plsc.subcore_barrier()
        pltpu.sync_copy(acc.at[pl.ds(sid * npw, npw)],
                        out_hbm.at[pl.ds(cid * n_pad + sid * npw, npw)])

    f = pl.kernel(
        body,
        out_type=jax.ShapeDtypeStruct((NC * n_pad,), jnp.float32),
        mesh=_sc_mesh(),
        scratch_types=[
            pltpu.VMEM((rows_w, CHUNK), jnp.int32),
            pltpu.VMEM((rows_w, CHUNK), jnp.float32),
            pltpu.VMEM((npw,), jnp.float32),
            pltpu.VMEM_SHARED((n_pad,), jnp.float32),
        ],
        name="sc_degree",
    )
    return f(col2d, ew2d)


def _sc_spmm(xs, row2d, col2d, ew2d, n_pad):
    d = xs.shape[1]
    ec = row2d.shape[0]
    rows_w = ec // NW
    nrw = n_pad // NS

    sb = 8
    nsb = rows_w // sb

    def body(xs_hbm, row_hbm, col_hbm, ew_hbm, out_hbm,
             erow, ecol, eew, gbuf, gsem, ssem, esem, acc):
        cid = lax.axis_index("c")
        sid = lax.axis_index("s")
        wid = cid * NS + sid

        def zero_body(i, _):
            for t in range(d // L):
                gbuf[0, i, pl.ds(t * L, L)] = jnp.zeros((L,), jnp.float32)
            return 0
        lax.fori_loop(0, CHUNK, zero_body, 0)
        for k in range(nrw // CHUNK):
            pltpu.sync_copy(gbuf.at[0],
                            acc.at[pl.ds(sid * nrw + k * CHUNK, CHUNK)])
        plsc.subcore_barrier()

        def eload(si, slot, sem):
            base = wid * rows_w + si * sb
            pltpu.async_copy(row_hbm.at[pl.ds(base, sb)], erow.at[slot], sem)
            pltpu.async_copy(col_hbm.at[pl.ds(base, sb)], ecol.at[slot], sem)
            pltpu.async_copy(ew_hbm.at[pl.ds(base, sb)], eew.at[slot], sem)

        def ewait(si, slot, sem):
            base = wid * rows_w + si * sb
            pltpu.make_async_copy(
                row_hbm.at[pl.ds(base, sb)], erow.at[slot], sem).wait()
            pltpu.make_async_copy(
                col_hbm.at[pl.ds(base, sb)], ecol.at[slot], sem).wait()
            pltpu.make_async_copy(
                ew_hbm.at[pl.ds(base, sb)], eew.at[slot], sem).wait()

        def gissue(slot, k, bi):
            for i in range(4):
                pltpu.async_copy(
                    xs_hbm.at[erow.at[slot, k, pl.ds(i * 32, 32)]],
                    gbuf.at[bi, pl.ds(i * 32, 32)], gsem)

        def gwait(slot, k, bi):
            for i in range(4):
                pltpu.make_async_copy(
                    xs_hbm.at[erow.at[slot, k, pl.ds(i * 32, 32)]],
                    gbuf.at[bi, pl.ds(i * 32, 32)], gsem).wait()

        eload(0, 0, esem)
        ewait(0, 0, esem)
        gissue(0, 0, 0)

        def chunk_body(j, _):
            b = lax.rem(j, 2)
            nb = 1 - b
            si = lax.div(j, sb)
            k = lax.rem(j, sb)
            slot = lax.rem(si, 2)

            gwait(slot, k, b)

            @pl.when(j >= 1)
            def _():
                jp = j - 1
                pltpu.make_async_copy(
                    gbuf.at[nb],
                    acc.at[ecol.at[lax.rem(lax.div(jp, sb), 2),
                                   lax.rem(jp, sb)]],
                    ssem).wait()

            @pl.when((k == 0) & (si + 1 < nsb))
            def _():
                eload(si + 1, 1 - slot, esem)

            @pl.when(j + 1 < rows_w)
            def _():
                jn = j + 1
                sin = lax.div(jn, sb)
                slotn = lax.rem(sin, 2)

                @pl.when(k == sb - 1)
                def _():
                    ewait(sin, slotn, esem)

                gissue(slotn, lax.rem(jn, sb), nb)

            def group_body(q, _):
                ewv = eew[slot, k, pl.ds(q * L, L)]
                for l in range(L):
                    e = q * L + l
                    s = _bcast_lane(ewv, l)
                    for t in range(d // L):
                        gbuf[b, e, pl.ds(t * L, L)] = (
                            gbuf[b, e, pl.ds(t * L, L)] * s)
                return 0
            lax.fori_loop(0, CHUNK // L, group_body, 0)

            pltpu.async_copy(gbuf.at[b], acc.at[ecol.at[slot, k]],
                             ssem, add=True)
            return 0
        lax.fori_loop(0, rows_w, chunk_body, 0)

        jl = rows_w - 1
        pltpu.make_async_copy(
            gbuf.at[lax.rem(jl, 2)],
            acc.at[ecol.at[lax.rem(lax.div(jl, sb), 2), lax.rem(jl, sb)]],
            ssem).wait()
        plsc.subcore_barrier()
        pltpu.sync_copy(acc.at[pl.ds(sid * nrw, nrw)],
                        out_hbm.at[cid, pl.ds(sid * nrw, nrw)])

    f = pl.kernel(
        body,
        out_type=jax.ShapeDtypeStruct((NC, n_pad, d), jnp.float32),
        mesh=_sc_mesh(),
        scratch_types=[
            pltpu.VMEM((2, sb, CHUNK), jnp.int32),
            pltpu.VMEM((2, sb, CHUNK), jnp.int32),
            pltpu.VMEM((2, sb, CHUNK), jnp.float32),
            pltpu.VMEM((2, CHUNK, d), jnp.float32),
            pltpu.SemaphoreType.DMA,
            pltpu.SemaphoreType.DMA,
            pltpu.SemaphoreType.DMA,
            pltpu.VMEM_SHARED((n_pad, d), jnp.float32),
        ],
        name=f"sc_spmm_d{d}",
    )
    return f(xs, row2d, col2d, ew2d)


def _tc_prep(features, w1, dega, degb, br=400):
    n, nf = features.shape
    nh = w1.shape[1]

    def body(f_ref, w_ref, d0_ref, d1_ref, xs_ref, dinv_ref):
        dinv = lax.rsqrt(1.0 + d0_ref[...] + d1_ref[...])
        x1 = jnp.dot(f_ref[...], w_ref[...], preferred_element_type=jnp.float32)
        xs_ref[...] = x1 * dinv
        dinv_ref[...] = dinv

    return pl.pallas_call(
        body,
        grid=(n // br,),
        in_specs=[
            pl.BlockSpec((br, nf), lambda i: (i, 0)),
            pl.BlockSpec((nf, nh), lambda i: (0, 0)),
            pl.BlockSpec((br, 1), lambda i: (i, 0)),
            pl.BlockSpec((br, 1), lambda i: (i, 0)),
        ],
        out_specs=[
            pl.BlockSpec((br, nh), lambda i: (i, 0)),
            pl.BlockSpec((br, 1), lambda i: (i, 0)),
        ],
        out_shape=[
            jax.ShapeDtypeStruct((n, nh), jnp.float32),
            jax.ShapeDtypeStruct((n, 1), jnp.float32),
        ],
        name="tc_prep",
    )(features, w1, dega, degb)


def _tc_mid(s1, xs1, dinv, b1, br=400):
    n, nh = xs1.shape

    def body(s_ref, xs_ref, dv_ref, b_ref, o_ref):
        s = s_ref[0] + s_ref[1] + xs_ref[...]
        h = jnp.maximum(dv_ref[...] * s + b_ref[...], 0.0)
        o_ref[...] = h * dv_ref[...]

    return pl.pallas_call(
        body,
        grid=(n // br,),
        in_specs=[
            pl.BlockSpec((2, br, nh), lambda i: (0, i, 0)),
            pl.BlockSpec((br, nh), lambda i: (i, 0)),
            pl.BlockSpec((br, 1), lambda i: (i, 0)),
            pl.BlockSpec((1, nh), lambda i: (0, 0)),
        ],
        out_specs=pl.BlockSpec((br, nh), lambda i: (i, 0)),
        out_shape=jax.ShapeDtypeStruct((n, nh), jnp.float32),
        name="tc_mid",
    )(s1, xs1, dinv, b1)


def _tc_final(s2, hs, dinv, w2, b2, br=400):
    n, nh = hs.shape
    ncls = w2.shape[1]

    def body(s_ref, hs_ref, dv_ref, w_ref, b_ref, o_ref):
        z = dv_ref[...] * (s_ref[0] + s_ref[1] + hs_ref[...])
        o = jnp.dot(z, w_ref[...], preferred_element_type=jnp.float32) + b_ref[...]
        m = jnp.max(o, axis=1, keepdims=True)
        lse = jnp.log(jnp.sum(jnp.exp(o - m), axis=1, keepdims=True))
        o_ref[...] = o - m - lse

    return pl.pallas_call(
        body,
        grid=(n // br,),
        in_specs=[
            pl.BlockSpec((2, br, nh), lambda i: (0, i, 0)),
            pl.BlockSpec((br, nh), lambda i: (i, 0)),
            pl.BlockSpec((br, 1), lambda i: (i, 0)),
            pl.BlockSpec((nh, ncls), lambda i: (0, 0)),
            pl.BlockSpec((1, ncls), lambda i: (0, 0)),
        ],
        out_specs=pl.BlockSpec((br, ncls), lambda i: (i, 0)),
        out_shape=jax.ShapeDtypeStruct((n, ncls), jnp.float32),
        name="tc_final",
    )(s2, hs, dinv, w2, b2)


@jax.jit
def kernel(features, edge_index, edge_weight, W1, b1, W2, b2):
    n, nf = features.shape
    nh = W1.shape[1]
    ncls = W2.shape[1]
    e = edge_index.shape[1]

    rows_w = -(-(-(-e // (NW * CHUNK))) // 8) * 8
    e_pad = rows_w * NW * CHUNK
    pad = e_pad - e
    row = jnp.concatenate([edge_index[0], jnp.zeros((pad,), edge_index.dtype)])
    col = jnp.concatenate([edge_index[1], jnp.zeros((pad,), edge_index.dtype)])
    ew = jnp.concatenate([edge_weight, jnp.zeros((pad,), edge_weight.dtype)])
    row2d = row.reshape(-1, CHUNK)
    col2d = col.reshape(-1, CHUNK)
    ew2d = ew.reshape(-1, CHUNK)

    n_pad = -(-n // (NS * CHUNK)) * (NS * CHUNK)

    degp = _sc_degree(col2d, ew2d, n_pad).reshape(NC, n_pad)
    dega = degp[0, :n].reshape(n, 1)
    degb = degp[1, :n].reshape(n, 1)

    xs1, dinv = _tc_prep(features, W1, dega, degb)
    s1 = _sc_spmm(xs1, row2d, col2d, ew2d, n_pad)

    hs = _tc_mid(s1, xs1, dinv, b1.reshape(1, nh))
    s2 = _sc_spmm(hs, row2d, col2d, ew2d, n_pad)
    return _tc_final(s2, hs, dinv, W2, b2.reshape(1, ncls))
